# baseline (speedup 1.0000x reference)
import numpy as np
import jax
import jax.numpy as jnp
from jax.experimental import pallas as pl

_LAYERS = [[1, 0, 20, 26, 25, 45],
           [0, 20, 12, 16, 2, 4, 8, 25, 45, 37, 41, 27, 29, 33],
           [12, 16, 2, 4, 8, 13, 17, 3, 5, 9, 3, 28, 37, 41, 27, 29, 33, 38, 42, 28, 30, 34],
           [13, 17, 3, 5, 9, 14, 18, 6, 10, 3, 28, 38, 42, 28, 30, 34, 39, 43, 31, 35],
           [14, 18, 6, 10, 15, 19, 7, 11, 39, 43, 31, 35, 40, 44, 32, 36],
           [15, 19, 7, 11, 21, 22, 23, 24, 40, 44, 32, 36, 46, 47, 48, 49]]

_S_np = np.zeros((6, 50), np.float32)
for _i, _lst in enumerate(_LAYERS):
    for _v in _lst:
        _S_np[_i, _v] += 1.0 / len(_lst)


def _max_body(x_ref, o_ref):
    m = jnp.max(x_ref[...], axis=3)
    o_ref[...] = m.reshape(o_ref.shape)


def _att_body(xt_ref, Wd_ref, bd_ref, g1_ref, b1_ref, A_ref, B_ref,
              g2_ref, b2_ref, Wagg_ref, bagg_ref, S_ref, out_ref):
    C, Nn, L, V = xt_ref.shape
    IC = Wd_ref.shape[0]
    xt2 = xt_ref[...].reshape(C, Nn * L * V)
    h = jnp.dot(Wd_ref[...], xt2, preferred_element_type=jnp.float32)
    h = h + bd_ref[...].reshape(IC, 1)
    mu = jnp.mean(h, axis=1, keepdims=True)
    var = jnp.mean((h - mu) ** 2, axis=1, keepdims=True)
    h = (h - mu) / jnp.sqrt(var + 1e-5)
    h = h * g1_ref[...].reshape(IC, 1) + b1_ref[...].reshape(IC, 1)
    h = jnp.maximum(h, 0.0)
    h4 = h.reshape(IC, Nn, L, V)
    xs = jnp.sum(h4 * S_ref[...][None, None, :, :], axis=3)
    prod = xs[:, :, :, None] * xs[:, :, None, :]
    raw = jnp.sum(prod, axis=0)
    xx = jnp.sum(xs * xs, axis=0)
    pd = 2.0 * raw - xx[:, :, None] - xx[:, None, :]
    iota = jax.lax.broadcasted_iota(jnp.int32, (Nn, L, L), 2)
    pdw = pd
    diffs = []
    for _ in range(3):
        m = jnp.max(pdw, axis=2, keepdims=True)
        idxv = jnp.min(jnp.where(pdw == m, iota, L), axis=2, keepdims=True)
        sel = iota == idxv
        pdw = jnp.where(sel, -1e30, pdw)
        oh = sel.astype(jnp.float32)
        fk = jnp.sum(oh[None, :, :, :] * xs[:, :, None, :], axis=3)
        diffs.append(fk - xs)
    xs2 = xs.reshape(IC, Nn * L)
    base = jnp.dot(B_ref[...], xs2, preferred_element_type=jnp.float32)
    es = [jnp.dot(A_ref[...], d.reshape(IC, Nn * L),
                  preferred_element_type=jnp.float32) + base for d in diffs]
    e = jnp.stack(es, axis=0)
    mu2 = jnp.mean(e, axis=(0, 2), keepdims=True)
    var2 = jnp.mean((e - mu2) ** 2, axis=(0, 2), keepdims=True)
    e = (e - mu2) / jnp.sqrt(var2 + 1e-5)
    e = e * g2_ref[...].reshape(1, IC, 1) + b2_ref[...].reshape(1, IC, 1)
    e = jnp.where(e > 0, e, 0.2 * e)
    att0 = jnp.max(e, axis=0)
    att = jnp.dot(Wagg_ref[...], att0, preferred_element_type=jnp.float32)
    att = att + bagg_ref[...].reshape(C, 1)
    g = jax.nn.sigmoid(att)
    out_ref[...] = g.reshape(C, Nn, L)


def _gate_body(x_ref, g_ref, o_ref):
    acc = x_ref[:, 0] * g_ref[:, 0][:, None, None]
    for l in range(1, x_ref.shape[1]):
        acc = acc + x_ref[:, l] * g_ref[:, l][:, None, None]
    o_ref[...] = acc


def kernel(x, W_down, b_down, gamma1, beta1, W_ec, gamma2, beta2, W_agg, b_agg):
    N, C, L, T, V = x.shape
    IC = W_down.shape[0]
    RB1 = 512

    def _max_dense(x_ref, o_ref):
        xb = x_ref[...]
        m = jnp.max(xb.reshape(RB1, T, V), axis=1)
        o_ref[...] = m

    xt_c = pl.pallas_call(
        _max_dense,
        grid=(N * C * L // RB1,),
        in_specs=[pl.BlockSpec((RB1, T * V), lambda r: (r, 0))],
        out_specs=pl.BlockSpec((RB1, V), lambda r: (r, 0)),
        out_shape=jax.ShapeDtypeStruct((N * C * L, V), x.dtype),
    )(x.reshape(N * C * L, T * V))
    return xt_c
    S = jnp.asarray(_S_np)
    A = W_ec[:, :IC]
    B = W_ec[:, IC:]
    g_c = pl.pallas_call(
        _att_body,
        out_shape=jax.ShapeDtypeStruct((C, N, L), jnp.float32),
    )(xt_c, W_down, b_down, gamma1, beta1, A, B, gamma2, beta2, W_agg, b_agg, S)
    gates = g_c.transpose(1, 0, 2).reshape(N * C, L)
    RB = 64
    out2 = pl.pallas_call(
        _gate_body,
        grid=(N * C // RB,),
        in_specs=[pl.BlockSpec((RB, L, T, V), lambda r: (r, 0, 0, 0)),
                  pl.BlockSpec((RB, L), lambda r: (r, 0))],
        out_specs=pl.BlockSpec((RB, T, V), lambda r: (r, 0, 0)),
        out_shape=jax.ShapeDtypeStruct((N * C, T, V), x.dtype),
    )(x.reshape(N * C, L, T, V), gates)
    return out2.reshape(N, C, T, V)


# speedup vs baseline: 6.4206x; 6.4206x over previous
import numpy as np
import jax
import jax.numpy as jnp
from jax.experimental import pallas as pl

_LAYERS = [[1, 0, 20, 26, 25, 45],
           [0, 20, 12, 16, 2, 4, 8, 25, 45, 37, 41, 27, 29, 33],
           [12, 16, 2, 4, 8, 13, 17, 3, 5, 9, 3, 28, 37, 41, 27, 29, 33, 38, 42, 28, 30, 34],
           [13, 17, 3, 5, 9, 14, 18, 6, 10, 3, 28, 38, 42, 28, 30, 34, 39, 43, 31, 35],
           [14, 18, 6, 10, 15, 19, 7, 11, 39, 43, 31, 35, 40, 44, 32, 36],
           [15, 19, 7, 11, 21, 22, 23, 24, 40, 44, 32, 36, 46, 47, 48, 49]]

_S_np = np.zeros((6, 50), np.float32)
for _i, _lst in enumerate(_LAYERS):
    for _v in _lst:
        _S_np[_i, _v] += 1.0 / len(_lst)
_SE_np = np.tile(_S_np, (16, 1))


def _max_body(x_ref, o_ref):
    o_ref[...] = jnp.max(x_ref[...], axis=1)


def _att_body(xt_ref, SE_ref, WdT_ref, bd_ref, g1_ref, b1_ref, AT_ref,
              BT_ref, g2_ref, b2_ref, WaggT_ref, bagg_ref, out_ref):
    R, C = xt_ref.shape
    IC = WdT_ref.shape[1]
    NL = SE_ref.shape[0]
    V = R // NL
    Nn = 16
    L = NL // Nn
    h = jnp.dot(xt_ref[...], WdT_ref[...], preferred_element_type=jnp.float32)
    h = h + bd_ref[...]
    mu = jnp.mean(h, axis=0, keepdims=True)
    var = jnp.mean((h - mu) ** 2, axis=0, keepdims=True)
    h = (h - mu) / jnp.sqrt(var + 1e-5)
    h = h * g1_ref[...] + b1_ref[...]
    h = jnp.maximum(h, 0.0)
    h3 = h.reshape(NL, V, IC)
    xs = jnp.sum(h3 * SE_ref[...][:, :, None], axis=1)
    xs3 = xs.reshape(Nn, L, IC)
    prod = xs3[:, :, None, :] * xs3[:, None, :, :]
    raw = jnp.sum(prod, axis=3)
    xx = jnp.sum(xs3 * xs3, axis=2)
    pd = 2.0 * raw - xx[:, :, None] - xx[:, None, :]
    iota = jax.lax.broadcasted_iota(jnp.int32, (Nn, L, L), 2)
    pdw = pd
    diffs = []
    for _ in range(3):
        m = jnp.max(pdw, axis=2, keepdims=True)
        idxv = jnp.min(jnp.where(pdw == m, iota, L), axis=2, keepdims=True)
        sel = iota == idxv
        pdw = jnp.where(sel, -1e30, pdw)
        oh = sel.astype(jnp.float32)
        fk = jnp.sum(oh[:, :, :, None] * xs3[:, None, :, :], axis=2)
        diffs.append((fk - xs3).reshape(NL, IC))
    base = jnp.dot(xs, BT_ref[...], preferred_element_type=jnp.float32)
    es = [jnp.dot(d, AT_ref[...], preferred_element_type=jnp.float32) + base
          for d in diffs]
    e = jnp.stack(es, axis=0)
    mu2 = jnp.mean(e, axis=(0, 1), keepdims=True)
    var2 = jnp.mean((e - mu2) ** 2, axis=(0, 1), keepdims=True)
    e = (e - mu2) / jnp.sqrt(var2 + 1e-5)
    e = e * g2_ref[...] + b2_ref[...]
    e = jnp.where(e > 0, e, 0.2 * e)
    att0 = jnp.max(e, axis=0)
    att = jnp.dot(att0, WaggT_ref[...], preferred_element_type=jnp.float32)
    att = att + bagg_ref[...]
    out_ref[...] = jax.nn.sigmoid(att)


def _gate_body(x_ref, g_ref, o_ref):
    acc = x_ref[0, 0] * g_ref[0, 0][None, :]
    for l in range(1, x_ref.shape[1]):
        acc = acc + x_ref[0, l] * g_ref[0, l][None, :]
    o_ref[0] = acc


def kernel(x, W_down, b_down, gamma1, beta1, W_ec, gamma2, beta2, W_agg, b_agg):
    N, C, L, T, V = x.shape
    IC = W_down.shape[0]
    y = jnp.transpose(x, (0, 2, 4, 3, 1))
    RB = 96
    xt = pl.pallas_call(
        _max_body,
        grid=(N * L * V // RB,),
        in_specs=[pl.BlockSpec((RB, T, C), lambda r: (r, 0, 0))],
        out_specs=pl.BlockSpec((RB, C), lambda r: (r, 0)),
        out_shape=jax.ShapeDtypeStruct((N * L * V, C), x.dtype),
    )(y.reshape(N * L * V, T, C))
    gates = pl.pallas_call(
        _att_body,
        out_shape=jax.ShapeDtypeStruct((N * L, C), jnp.float32),
    )(xt, jnp.asarray(_SE_np), W_down.T, b_down, gamma1, beta1,
      W_ec[:, :IC].T, W_ec[:, IC:].T, gamma2, beta2, W_agg.T, b_agg)
    MB = 1600
    out_y = pl.pallas_call(
        _gate_body,
        grid=(N, V * T // MB),
        in_specs=[pl.BlockSpec((1, L, MB, C), lambda n, mb: (n, 0, mb, 0)),
                  pl.BlockSpec((1, L, C), lambda n, mb: (n, 0, 0))],
        out_specs=pl.BlockSpec((1, MB, C), lambda n, mb: (n, mb, 0)),
        out_shape=jax.ShapeDtypeStruct((N, V * T, C), x.dtype),
    )(y.reshape(N, L, V * T, C), gates.reshape(N, L, C))
    return jnp.transpose(out_y.reshape(N, V, T, C), (0, 3, 2, 1))
